# baseline (speedup 1.0000x reference)
import functools
import numpy as np
import jax
import jax.numpy as jnp
from jax import lax
from jax.experimental import pallas as pl
from jax.experimental.pallas import tpu as pltpu, tpu_sc as plsc

B, N, C = 8, 20000, 90
K_TOP = 5000
MAX_DET = 100
IOU_THR = 0.5

NTILE = 16
ROWS_T = 1248
NCH = 6
CHR = 208
TAIL_R0 = NTILE * ROWS_T
TAILR = 32
CAP_T = 1024
CAP = 8192
NDEC = 5120

U31 = np.uint32(0x80000000)
UFF = np.uint32(0xFFFFFFFF)


def _f32_to_key(xv):
    u = plsc.bitcast(xv, jnp.uint32)
    neg = (u >> jnp.uint32(31)) == jnp.uint32(1)
    return jnp.where(neg, u ^ UFF, u ^ U31)


def _key_to_f32(k):
    pos = (k >> jnp.uint32(31)) == jnp.uint32(1)
    u = jnp.where(pos, k ^ U31, k ^ UFF)
    return plsc.bitcast(u, jnp.float32)


def _store1(ref, pos, val, iota):
    plsc.store_scatter(ref, [jnp.full((16,), pos, jnp.int32)],
                       jnp.full((16,), val), mask=iota == 0)


def _body_a(cls_hbm, sk_hbm, ix_hbm, ct_hbm,
            chunk, chunk2, sem0, sem1, hist, stripe, gsh, btl, btf, blockb,
            skb, idxb, cnt16, hist_sp, ghist_sp, btot_sp):
    c = lax.axis_index("c")
    s = lax.axis_index("s")
    iota = lax.iota(jnp.int32, 16)
    zero16 = jnp.zeros((16,), jnp.int32)
    occ_c, _ = plsc.scan_count(zero16)
    bias = jnp.max(occ_c) - 15
    v6 = iota < 10

    bufs2 = [chunk, chunk2]
    sems2 = [sem0, sem1]

    def phase_a(bi, _):
        b = 4 * c + bi

        def zh(j, _):
            hist[pl.ds(16 * j, 16)] = zero16
            return 0
        lax.fori_loop(0, 512, zh, 0)

        def row_vecs(buf, r, fn):
            for j in range(5):
                fn(buf[r, pl.ds(16 * j, 16)], 16 * j, None)
            fn(buf[r, pl.ds(80, 16)], 80, v6)

        def hist_fn(xv, coff, valid):
            k = _f32_to_key(xv)
            bkt = (k >> jnp.uint32(19)).astype(jnp.int32)
            occ, lastm = plsc.scan_count(bkt, mask=valid)
            plsc.addupdate_scatter(hist, [bkt], occ - bias + 1, mask=lastm)

        descs = [None, None]
        descs[0] = pltpu.async_copy(
            cls_hbm.at[b, pl.ds(s * ROWS_T, CHR), :], chunk, sem0)
        for ch in range(NCH):
            if ch + 1 < NCH:
                r0 = s * ROWS_T + (ch + 1) * CHR
                descs[(ch + 1) % 2] = pltpu.async_copy(
                    cls_hbm.at[b, pl.ds(r0, CHR), :], bufs2[(ch + 1) % 2],
                    sems2[(ch + 1) % 2])
            descs[ch % 2].wait()
            buf = bufs2[ch % 2]

            @plsc.parallel_loop(0, CHR, unroll=2)
            def _(r):
                row_vecs(buf, r, hist_fn)

        @pl.when(s == NTILE - 1)
        def _():
            pltpu.sync_copy(cls_hbm.at[b, pl.ds(TAIL_R0, TAILR), :],
                            chunk.at[pl.ds(0, TAILR), :])

            @plsc.parallel_loop(0, TAILR, unroll=2)
            def _(r):
                row_vecs(chunk, r, hist_fn)

        pltpu.sync_copy(hist, hist_sp.at[s])
        plsc.subcore_barrier()

        for h in range(2):
            off = s * 512 + h * 256
            pltpu.sync_copy(hist_sp.at[:, pl.ds(off, 256)], stripe)

            def red(g, _):
                acc = zero16
                for r in range(16):
                    acc = acc + stripe[r, pl.ds(16 * g, 16)]
                gsh[pl.ds(16 * g, 16)] = acc
                _store1(btl, h * 16 + g, jnp.sum(acc), iota)
                return 0
            lax.fori_loop(0, 16, red, 0)
            pltpu.sync_copy(gsh, ghist_sp.at[pl.ds(off, 256)])
        pltpu.sync_copy(btl, btot_sp.at[pl.ds(s * 32, 32)])
        plsc.subcore_barrier()

        pltpu.sync_copy(btot_sp, btf)

        def scan(v, carry):
            running, found, bb, run_at = carry
            grp = btf[pl.ds(496 - 16 * v, 16)]
            rv = lax.rev(grp, (0,))
            cs = plsc.cumsum(rv)
            tot = jnp.max(cs)
            ge = (running + cs) >= K_TOP
            j = jnp.max(plsc.all_reduce_ffs(ge))
            crossed = jnp.logical_and(found == 0, (running + tot) >= K_TOP)
            pre = jnp.max(jnp.where(iota < j, cs, 0))
            bb = jnp.where(crossed, (496 - 16 * v) + 15 - j, bb)
            run_at = jnp.where(crossed, running + pre, run_at)
            found = jnp.where(crossed, 1, found)
            return running + tot, found, bb, run_at

        _, _, bb, run_at = lax.fori_loop(
            0, 32, scan, (jnp.int32(0), jnp.int32(0), jnp.int32(0), jnp.int32(0)))

        pltpu.sync_copy(ghist_sp.at[pl.ds(16 * bb, 16)], blockb)
        rv = lax.rev(blockb[...], (0,))
        cs = plsc.cumsum(rv)
        ge = (run_at + cs) >= K_TOP
        j2 = jnp.max(plsc.all_reduce_ffs(ge))
        bstar = 16 * bb + 15 - jnp.minimum(j2, 15)
        t_u = bstar.astype(jnp.uint32) << jnp.uint32(19)

        def col_fn(cnt, xv, gbase, valid):
            k = _f32_to_key(xv)
            msel = k >= t_u
            if valid is not None:
                msel = jnp.logical_and(msel, valid)
            nv = jnp.max(plsc.all_reduce_population_count(msel))
            plsc.store_compressed(skb.at[pl.ds(cnt, 16)],
                                  plsc.bitcast(k, jnp.int32), mask=msel)
            plsc.store_compressed(idxb.at[pl.ds(cnt, 16)],
                                  gbase + iota, mask=msel)
            return jnp.minimum(cnt + nv, CAP_T - 16)

        def col_row(buf, r, rowbase, cnt):
            for j in range(5):
                cnt = col_fn(cnt, buf[r, pl.ds(16 * j, 16)],
                             rowbase + 16 * j, None)
            return col_fn(cnt, buf[r, pl.ds(80, 16)], rowbase + 80, v6)

        descs[0] = pltpu.async_copy(
            cls_hbm.at[b, pl.ds(s * ROWS_T, CHR), :], chunk, sem0)
        cnt = jnp.int32(0)
        for ch in range(NCH):
            if ch + 1 < NCH:
                r0 = s * ROWS_T + (ch + 1) * CHR
                descs[(ch + 1) % 2] = pltpu.async_copy(
                    cls_hbm.at[b, pl.ds(r0, CHR), :], bufs2[(ch + 1) % 2],
                    sems2[(ch + 1) % 2])
            descs[ch % 2].wait()
            buf = bufs2[ch % 2]
            rbase = s * ROWS_T + ch * CHR

            @plsc.parallel_loop(0, CHR, unroll=2, carry=cnt)
            def cnt(r, cnt):
                return col_row(buf, r, (rbase + r) * 90, cnt)

        @pl.when(s == NTILE - 1)
        def _():
            pltpu.sync_copy(cls_hbm.at[b, pl.ds(TAIL_R0, TAILR), :],
                            chunk.at[pl.ds(0, TAILR), :])

            @plsc.parallel_loop(0, TAILR, unroll=2, carry=cnt)
            def cnt2(r, c2):
                return col_row(chunk, r, (TAIL_R0 + r) * 90, c2)
            cnt16[...] = jnp.full((16,), cnt2, jnp.int32)

        @pl.when(s != NTILE - 1)
        def _():
            cnt16[...] = jnp.full((16,), cnt, jnp.int32)

        base = (b * 16 + s)
        pltpu.sync_copy(skb, sk_hbm.at[pl.ds(base * CAP_T, CAP_T)])
        pltpu.sync_copy(idxb, ix_hbm.at[pl.ds(base * CAP_T, CAP_T)])
        pltpu.sync_copy(cnt16, ct_hbm.at[pl.ds(base * 16, 16)])
        return 0

    lax.fori_loop(0, 4, phase_a, 0)


def _body_b(sk_hbm, ix_hbm, ct_hbm, comb_hbm, scl_hbm, out_hbm,
            cbuf, tmpk, tmpi, k0, i0, k1, i1, h256,
            gidc, scoreb, y1b, x1b, y2b, x2b,
            combrow, sclv, ay1, ax1, ay2, ax2, aar, outb):
    c = lax.axis_index("c")
    s = lax.axis_index("s")
    iota = lax.iota(jnp.int32, 16)
    zero16 = jnp.zeros((16,), jnp.int32)
    occ_c, _ = plsc.scan_count(zero16)
    bias = jnp.max(occ_c) - 15

    @pl.when(s < 4)
    def _():
        b = 4 * c + s
        pltpu.sync_copy(scl_hbm, sclv)
        pltpu.sync_copy(ct_hbm.at[pl.ds(b * 256, 256)], cbuf)

        def pack_t(t, off):
            base = (b * 16 + t) * CAP_T
            pltpu.sync_copy(sk_hbm.at[pl.ds(base, CAP_T)], tmpk)
            pltpu.sync_copy(ix_hbm.at[pl.ds(base, CAP_T)], tmpi)
            cntt = cbuf[pl.ds(16 * t, 16)][0]
            iters = (cntt + 15) // 16

            def pv(v, off):
                nv = jnp.clip(jnp.minimum(cntt - 16 * v,
                                          jnp.minimum(16, (CAP - 16) - off)), 0, 16)
                msk = iota < nv
                kv = tmpk[pl.ds(16 * v, 16)]
                plsc.store_compressed(k0.at[pl.ds(off, 16)],
                                      kv ^ np.int32(-1), mask=msk)
                iv = tmpi[pl.ds(16 * v, 16)]
                plsc.store_compressed(i0.at[pl.ds(off, 16)], iv, mask=msk)
                return off + nv
            return lax.fori_loop(0, iters, pv, off)

        total = lax.fori_loop(0, NTILE, pack_t, jnp.int32(0))
        plsc.store_scatter(k0, [total + iota], jnp.full((16,), -1, jnp.int32))
        plsc.store_scatter(i0, [total + iota], zero16)
        nvec = (total + 15) // 16

        bufs = [(k0, i0), (k1, i1)]
        for p in range(4):
            kin, iin = bufs[p % 2]
            kout, iout = bufs[(p + 1) % 2]
            sh = jnp.uint32(8 * p)

            def zh(j, _):
                h256[pl.ds(16 * j, 16)] = zero16
                return 0
            lax.fori_loop(0, 16, zh, 0)

            def hv(v, _):
                k = plsc.bitcast(kin[pl.ds(16 * v, 16)], jnp.uint32)
                d = ((k >> sh) & jnp.uint32(255)).astype(jnp.int32)
                occ, lastm = plsc.scan_count(d)
                plsc.addupdate_scatter(h256, [d], occ - bias + 1, mask=lastm)
                return 0
            lax.fori_loop(0, nvec, hv, 0)

            def pfx(j, run):
                h = h256[pl.ds(16 * j, 16)]
                cs = plsc.cumsum(h)
                h256[pl.ds(16 * j, 16)] = run + cs - h
                return run + jnp.max(cs)
            lax.fori_loop(0, 16, pfx, jnp.int32(0))

            def sv(v, _):
                k = kin[pl.ds(16 * v, 16)]
                idx = iin[pl.ds(16 * v, 16)]
                d = ((plsc.bitcast(k, jnp.uint32) >> sh)
                     & jnp.uint32(255)).astype(jnp.int32)
                occ, lastm = plsc.scan_count(d)
                occ0 = occ - bias
                base = plsc.load_gather(h256, [d])
                dest = base + occ0
                plsc.store_scatter(kout, [dest], k)
                plsc.store_scatter(iout, [dest], idx)
                plsc.addupdate_scatter(h256, [d], occ0 + 1, mask=lastm)
                return 0
            lax.fori_loop(0, nvec, sv, 0)

        def dv(v, _):
            k = plsc.bitcast(k0[pl.ds(16 * v, 16)] ^ np.int32(-1), jnp.uint32)
            val = _key_to_f32(k)
            scoreb[pl.ds(16 * v, 16)] = 1.0 / (1.0 + jnp.exp(-val))
            return 0
        lax.fori_loop(0, NDEC // 16, dv, 0)

        for cc in range(8):
            def gi(u, _):
                idx = i0[pl.ds(640 * cc + 16 * u, 16)]
                qf = (idx.astype(jnp.float32) * jnp.float32(1.0 / 90.0)
                      + jnp.float32(0.5))
                q = qf.astype(jnp.int32)
                r = idx - q * 90
                q = jnp.minimum(jnp.where(r < 0, q - 1, q), N - 1)
                gidc[pl.ds(16 * u, 16)] = q + N * b
                return 0
            lax.fori_loop(0, 40, gi, 0)
            pltpu.sync_copy(comb_hbm.at[gidc], combrow)

            def dec(u, _):
                rid = iota + 16 * u
                c0 = jnp.zeros((16,), jnp.int32)
                ty = plsc.load_gather(combrow, [rid, c0])
                tx = plsc.load_gather(combrow, [rid, c0 + 1])
                th = plsc.load_gather(combrow, [rid, c0 + 2])
                tw = plsc.load_gather(combrow, [rid, c0 + 3])
                a0 = plsc.load_gather(combrow, [rid, c0 + 4])
                a1 = plsc.load_gather(combrow, [rid, c0 + 5])
                a2 = plsc.load_gather(combrow, [rid, c0 + 6])
                a3 = plsc.load_gather(combrow, [rid, c0 + 7])
                yca = (a0 + a2) * 0.5
                xca = (a1 + a3) * 0.5
                ha = a2 - a0
                wa = a3 - a1
                ww = jnp.exp(tw) * wa
                hh = jnp.exp(th) * ha
                yc = ty * ha + yca
                xc = tx * wa + xca
                y1 = yc - hh * 0.5
                x1 = xc - ww * 0.5
                y2 = yc + hh * 0.5
                x2 = xc + ww * 0.5
                o = pl.ds(640 * cc + 16 * u, 16)
                y1b[o] = y1
                x1b[o] = x1
                y2b[o] = y2
                x2b[o] = x2
                return 0
            lax.fori_loop(0, 40, dec, 0)

        def initacc(j, _):
            far = jnp.full((16,), 3e8, jnp.float32)
            ay1[pl.ds(16 * j, 16)] = far
            ax1[pl.ds(16 * j, 16)] = far
            ay2[pl.ds(16 * j, 16)] = far
            ax2[pl.ds(16 * j, 16)] = far
            aar[pl.ds(16 * j, 16)] = jnp.zeros((16,), jnp.float32)
            return 0
        lax.fori_loop(0, 7, initacc, 0)

        def initout(j, _):
            outb[pl.ds(16 * j, 16)] = jnp.zeros((16,), jnp.float32)
            return 0
        lax.fori_loop(0, 38, initout, 0)

        scale = jnp.max(plsc.load_gather(
            sclv, [jnp.full((16,), b, jnp.int32)]))
        bound = jnp.minimum(jnp.int32(K_TOP), total)

        def nms_cond(carry):
            i, n = carry
            return jnp.logical_and(n < MAX_DET, i < bound)

        def nms_body(carry):
            i, n = carry
            cy1 = y1b[pl.ds(i, 16)][0]
            cx1 = x1b[pl.ds(i, 16)][0]
            cy2 = y2b[pl.ds(i, 16)][0]
            cx2 = x2b[pl.ds(i, 16)][0]
            car = (cy2 - cy1) * (cx2 - cx1)

            def iou_a(j, mx):
                o = pl.ds(16 * j, 16)
                yA = jnp.maximum(jnp.full((16,), cy1), ay1[o])
                xA = jnp.maximum(jnp.full((16,), cx1), ax1[o])
                yB = jnp.minimum(jnp.full((16,), cy2), ay2[o])
                xB = jnp.minimum(jnp.full((16,), cx2), ax2[o])
                inter = jnp.maximum(yB - yA, 0.0) * jnp.maximum(xB - xA, 0.0)
                iou = inter / (car + aar[o] - inter + 1e-8)
                return jnp.maximum(mx, jnp.max(iou))

            mx = lax.fori_loop(0, (n + 15) // 16, iou_a, jnp.float32(0.0))
            keep = mx <= IOU_THR

            @pl.when(keep)
            def _():
                _store1(ay1, n, cy1, iota)
                _store1(ax1, n, cx1, iota)
                _store1(ay2, n, cy2, iota)
                _store1(ax2, n, cx2, iota)
                _store1(aar, n, car, iota)
                idx_s = i0[pl.ds(i, 16)][0]
                qf = (idx_s.astype(jnp.float32) * jnp.float32(1.0 / 90.0)
                      + jnp.float32(0.5))
                q = qf.astype(jnp.int32)
                rr = idx_s - q * 90
                q = jnp.where(rr < 0, q - 1, q)
                cls = idx_s - 90 * q
                _store1(outb, n * 6 + 0, cy1 * scale, iota)
                _store1(outb, n * 6 + 1, cx1 * scale, iota)
                _store1(outb, n * 6 + 2, cy2 * scale, iota)
                _store1(outb, n * 6 + 3, cx2 * scale, iota)
                _store1(outb, n * 6 + 4, scoreb[pl.ds(i, 16)][0], iota)
                _store1(outb, n * 6 + 5, (cls + 1).astype(jnp.float32), iota)

            return i + 1, n + keep.astype(jnp.int32)

        lax.while_loop(nms_cond, nms_body, (jnp.int32(0), jnp.int32(0)))
        pltpu.sync_copy(outb.at[pl.ds(0, 600)], out_hbm.at[b])


@jax.jit
def kernel(cls_out, box_out, anchors, image_scales):
    mesh = plsc.VectorSubcoreMesh(core_axis_name="c", subcore_axis_name="s")
    fa = pl.kernel(
        _body_a,
        out_type=[jax.ShapeDtypeStruct((B * 16 * CAP_T,), jnp.int32),
                  jax.ShapeDtypeStruct((B * 16 * CAP_T,), jnp.int32),
                  jax.ShapeDtypeStruct((B * 16 * 16,), jnp.int32)],
        mesh=mesh,
        compiler_params=pltpu.CompilerParams(needs_layout_passes=False,
                                             use_tc_tiling_on_sc=True),
        scratch_types=[
            pltpu.VMEM((CHR, 128), jnp.float32),
            pltpu.VMEM((CHR, 128), jnp.float32),
            pltpu.SemaphoreType.DMA,
            pltpu.SemaphoreType.DMA,
            pltpu.VMEM((8192,), jnp.int32),
            pltpu.VMEM((16, 256), jnp.int32),
            pltpu.VMEM((256,), jnp.int32),
            pltpu.VMEM((32,), jnp.int32),
            pltpu.VMEM((512,), jnp.int32),
            pltpu.VMEM((16,), jnp.int32),
            pltpu.VMEM((CAP_T,), jnp.int32),
            pltpu.VMEM((CAP_T,), jnp.int32),
            pltpu.VMEM((16,), jnp.int32),
            pltpu.VMEM_SHARED((16, 8192), jnp.int32),
            pltpu.VMEM_SHARED((8192,), jnp.int32),
            pltpu.VMEM_SHARED((512,), jnp.int32),
        ],
    )
    fb = pl.kernel(
        _body_b,
        out_type=jax.ShapeDtypeStruct((B, 600), jnp.float32),
        mesh=mesh,
        compiler_params=pltpu.CompilerParams(needs_layout_passes=False,
                                             use_tc_tiling_on_sc=False),
        scratch_types=[
            pltpu.VMEM((256,), jnp.int32),
            pltpu.VMEM((CAP_T,), jnp.int32),
            pltpu.VMEM((CAP_T,), jnp.int32),
            pltpu.VMEM((CAP,), jnp.int32),
            pltpu.VMEM((CAP,), jnp.int32),
            pltpu.VMEM((CAP,), jnp.int32),
            pltpu.VMEM((CAP,), jnp.int32),
            pltpu.VMEM((256,), jnp.int32),
            pltpu.VMEM((640,), jnp.int32),
            pltpu.VMEM((NDEC,), jnp.float32),
            pltpu.VMEM((NDEC,), jnp.float32),
            pltpu.VMEM((NDEC,), jnp.float32),
            pltpu.VMEM((NDEC,), jnp.float32),
            pltpu.VMEM((NDEC,), jnp.float32),
            pltpu.VMEM((640, 8), jnp.float32),
            pltpu.VMEM((16,), jnp.float32),
            pltpu.VMEM((112,), jnp.float32),
            pltpu.VMEM((112,), jnp.float32),
            pltpu.VMEM((112,), jnp.float32),
            pltpu.VMEM((112,), jnp.float32),
            pltpu.VMEM((112,), jnp.float32),
            pltpu.VMEM((608,), jnp.float32),
        ],
    )
    cls_pad = jnp.pad(cls_out, ((0, 0), (0, 0), (0, 128 - C)))
    sk, ix, ct = fa(cls_pad)
    comb = jnp.concatenate(
        [box_out, jnp.broadcast_to(anchors[None], (B, N, 4))], axis=2
    ).reshape(B * N, 8)
    scl16 = jnp.pad(image_scales, (0, 16 - B))
    out = fb(sk, ix, ct, comb, scl16)
    return out.reshape(B, MAX_DET, 6)


# speedup vs baseline: 1.5675x; 1.5675x over previous
import functools
import jax
import jax.numpy as jnp
from jax import lax
from jax.experimental import pallas as pl
from jax.experimental.pallas import tpu as pltpu, tpu_sc as plsc

B, N, C = 8, 20000, 90
NC = N * C
K_TOP = 5000
MAX_DET = 100
IOU_THR = 0.5

NTILE = 16
SLICE = 112320
NCH = 12
CH = 9360
CHV = 585
TAILV = 180
TAIL_OFF = NTILE * SLICE
CAP_T = 1024
CAP = 8192
NDEC = 5120
NB_PER_CORE = 4

import numpy as np

U31 = np.uint32(0x80000000)
UFF = np.uint32(0xFFFFFFFF)


def _f32_to_key(xv):
    u = plsc.bitcast(xv, jnp.uint32)
    neg = (u >> jnp.uint32(31)) == jnp.uint32(1)
    return jnp.where(neg, u ^ UFF, u ^ U31)


def _key_to_f32(k):
    pos = (k >> jnp.uint32(31)) == jnp.uint32(1)
    u = jnp.where(pos, k ^ U31, k ^ UFF)
    return plsc.bitcast(u, jnp.float32)


def _store1(ref, pos, val, iota):
    plsc.store_scatter(ref, [jnp.full((16,), pos, jnp.int32)],
                       jnp.full((16,), val), mask=iota == 0)


def _sc_body(cls_hbm, comb_hbm, scl_hbm, out_hbm,
             chunk, chunk2, sem0, sem1, hist, stripe, gsh, btl, btf, blockb,
             skb, idxb, cnt16,
             cbuf, tmpk, tmpi, k0, i0, k1, i1, h256,
             gidc, scoreb, y1b, x1b, y2b, x2b,
             combrow, sclv, ay1, ax1, ay2, ax2, aar, outb,
             hist_sp, ghist_sp, btot_sp, cskey_sp, cidx_sp, cnts_sp):
    c = lax.axis_index("c")
    s = lax.axis_index("s")
    iota = lax.iota(jnp.int32, 16)
    zero16 = jnp.zeros((16,), jnp.int32)

    occ_c, _ = plsc.scan_count(zero16)
    bias = jnp.max(occ_c) - 15

    pltpu.sync_copy(scl_hbm, sclv)

    def phase_a(bi, _):
        b = 4 * c + bi
        row0 = b * NC
        bufs2 = [chunk, chunk2]
        sems2 = [sem0, sem1]

        def hist_vec(buf, v):
            xv = buf[pl.ds(16 * v, 16)]
            k = _f32_to_key(xv)
            bkt = (k >> jnp.uint32(19)).astype(jnp.int32)
            occ, lastm = plsc.scan_count(bkt)
            plsc.addupdate_scatter(hist, [bkt], occ - bias + 1, mask=lastm)

        def hist_threshold(nch_hist, with_tail, target):
            def zh(j, _):
                hist[pl.ds(16 * j, 16)] = zero16
                return 0
            lax.fori_loop(0, 512, zh, 0)

            descs = [None, None]
            descs[0] = pltpu.async_copy(
                cls_hbm.at[pl.ds(row0 + s * SLICE, CH)], chunk, sem0)
            for ch in range(nch_hist):
                if ch + 1 < nch_hist:
                    start = row0 + s * SLICE + (ch + 1) * CH
                    descs[(ch + 1) % 2] = pltpu.async_copy(
                        cls_hbm.at[pl.ds(start, CH)], bufs2[(ch + 1) % 2],
                        sems2[(ch + 1) % 2])
                descs[ch % 2].wait()
                buf = bufs2[ch % 2]

                @plsc.parallel_loop(0, CHV, unroll=8)
                def _(v):
                    hist_vec(buf, v)

            if with_tail:
                @pl.when(s == NTILE - 1)
                def _():
                    pltpu.sync_copy(cls_hbm.at[pl.ds(row0 + TAIL_OFF, 2880)],
                                    chunk.at[pl.ds(0, 2880)])

                    @plsc.parallel_loop(0, TAILV, unroll=4)
                    def _(v):
                        hist_vec(chunk, v)

            pltpu.sync_copy(hist, hist_sp.at[s])
            plsc.subcore_barrier()

            for h in range(2):
                off = s * 512 + h * 256
                pltpu.sync_copy(hist_sp.at[:, pl.ds(off, 256)], stripe)

                def red(g, _):
                    acc = zero16
                    for r in range(16):
                        acc = acc + stripe[r, pl.ds(16 * g, 16)]
                    gsh[pl.ds(16 * g, 16)] = acc
                    _store1(btl, h * 16 + g, jnp.sum(acc), iota)
                    return 0
                lax.fori_loop(0, 16, red, 0)
                pltpu.sync_copy(gsh, ghist_sp.at[pl.ds(off, 256)])
            pltpu.sync_copy(btl, btot_sp.at[pl.ds(s * 32, 32)])
            plsc.subcore_barrier()

            pltpu.sync_copy(btot_sp, btf)

            def scan(v, carry):
                running, found, bb, run_at = carry
                grp = btf[pl.ds(496 - 16 * v, 16)]
                rv = lax.rev(grp, (0,))
                cs = plsc.cumsum(rv)
                tot = jnp.max(cs)
                ge = (running + cs) >= target
                j = jnp.max(plsc.all_reduce_ffs(ge))
                crossed = jnp.logical_and(found == 0, (running + tot) >= target)
                pre = jnp.max(jnp.where(iota < j, cs, 0))
                bb = jnp.where(crossed, (496 - 16 * v) + 15 - j, bb)
                run_at = jnp.where(crossed, running + pre, run_at)
                found = jnp.where(crossed, 1, found)
                return running + tot, found, bb, run_at

            _, _, bb, run_at = lax.fori_loop(
                0, 32, scan,
                (jnp.int32(0), jnp.int32(0), jnp.int32(0), jnp.int32(0)))

            pltpu.sync_copy(ghist_sp.at[pl.ds(16 * bb, 16)], blockb)
            rv = lax.rev(blockb[...], (0,))
            cs = plsc.cumsum(rv)
            ge = (run_at + cs) >= target
            j2 = jnp.max(plsc.all_reduce_ffs(ge))
            bstar = 16 * bb + 15 - jnp.minimum(j2, 15)
            return bstar.astype(jnp.uint32) << jnp.uint32(19)

        def collect(t_u):
            def col_vec(cnt, buf, v, gbase0):
                xv = buf[pl.ds(16 * v, 16)]
                k = _f32_to_key(xv)
                msel = k >= t_u
                nv = jnp.max(plsc.all_reduce_population_count(msel))
                plsc.store_compressed(skb.at[pl.ds(cnt, 16)],
                                      plsc.bitcast(k, jnp.int32), mask=msel)
                plsc.store_compressed(idxb.at[pl.ds(cnt, 16)],
                                      gbase0 + 16 * v + iota, mask=msel)
                return jnp.minimum(cnt + nv, CAP_T - 16)

            descs = [None, None]
            descs[0] = pltpu.async_copy(
                cls_hbm.at[pl.ds(row0 + s * SLICE, CH)], chunk, sem0)
            cnt = jnp.int32(0)
            for ch in range(NCH):
                if ch + 1 < NCH:
                    start = row0 + s * SLICE + (ch + 1) * CH
                    descs[(ch + 1) % 2] = pltpu.async_copy(
                        cls_hbm.at[pl.ds(start, CH)], bufs2[(ch + 1) % 2],
                        sems2[(ch + 1) % 2])
                descs[ch % 2].wait()
                buf = bufs2[ch % 2]
                gbase0 = s * SLICE + ch * CH

                @plsc.parallel_loop(0, CHV, unroll=8, carry=cnt)
                def cnt(v, cnt):
                    return col_vec(cnt, buf, v, gbase0)

            @pl.when(s == NTILE - 1)
            def _():
                pltpu.sync_copy(cls_hbm.at[pl.ds(row0 + TAIL_OFF, 2880)],
                                chunk.at[pl.ds(0, 2880)])

                @plsc.parallel_loop(0, TAILV, unroll=4, carry=cnt)
                def cnt2(v, c2):
                    return col_vec(c2, chunk, v, TAIL_OFF)
                cnt16[...] = jnp.full((16,), cnt2, jnp.int32)

            @pl.when(s != NTILE - 1)
            def _():
                cnt16[...] = jnp.full((16,), cnt, jnp.int32)

            pltpu.sync_copy(skb, cskey_sp.at[bi, s])
            pltpu.sync_copy(idxb, cidx_sp.at[bi, s])
            pltpu.sync_copy(cnt16, cnts_sp.at[bi, s])
            plsc.subcore_barrier()

        t_est = hist_threshold(1, False, 550)
        collect(t_est)

        pltpu.sync_copy(cnts_sp.at[bi], stripe.at[:, pl.ds(0, 16)])
        cnts = plsc.load_gather(stripe, [iota, zero16])
        total = jnp.sum(cnts)
        bad = jnp.logical_or(total < K_TOP, jnp.max(cnts) >= CAP_T - 16)

        @pl.when(bad)
        def _():
            t_ex = hist_threshold(NCH, True, K_TOP)
            collect(t_ex)
        return 0

    lax.fori_loop(0, NB_PER_CORE, phase_a, 0)

    @pl.when(s < NB_PER_CORE)
    def _():
        slot = s
        b = 4 * c + slot

        pltpu.sync_copy(cnts_sp.at[slot], cbuf)

        def pack_t(t, off):
            pltpu.sync_copy(cskey_sp.at[slot, t], tmpk)
            pltpu.sync_copy(cidx_sp.at[slot, t], tmpi)
            cntt = jnp.max(plsc.load_gather(
                cbuf, [jnp.full((16,), t, jnp.int32), zero16]))
            iters = (cntt + 15) // 16

            def pv(v, off):
                nv = jnp.clip(jnp.minimum(cntt - 16 * v,
                                          jnp.minimum(16, (CAP - 16) - off)), 0, 16)
                msk = iota < nv
                kv = tmpk[pl.ds(16 * v, 16)]
                plsc.store_compressed(k0.at[pl.ds(off, 16)],
                                      kv ^ np.int32(-1), mask=msk)
                iv = tmpi[pl.ds(16 * v, 16)]
                plsc.store_compressed(i0.at[pl.ds(off, 16)], iv, mask=msk)
                return off + nv
            return lax.fori_loop(0, iters, pv, off)

        total = lax.fori_loop(0, NTILE, pack_t, jnp.int32(0))
        plsc.store_scatter(k0, [total + iota], jnp.full((16,), -1, jnp.int32))
        plsc.store_scatter(i0, [total + iota], zero16)
        nvec = (total + 15) // 16

        bufs = [(k0, i0), (k1, i1)]
        for p in range(4):
            kin, iin = bufs[p % 2]
            kout, iout = bufs[(p + 1) % 2]
            sh = jnp.uint32(8 * p)

            def zh(j, _):
                h256[pl.ds(16 * j, 16)] = zero16
                return 0
            lax.fori_loop(0, 16, zh, 0)

            def hv(v, _):
                k = plsc.bitcast(kin[pl.ds(16 * v, 16)], jnp.uint32)
                d = ((k >> sh) & jnp.uint32(255)).astype(jnp.int32)
                occ, lastm = plsc.scan_count(d)
                plsc.addupdate_scatter(h256, [d], occ - bias + 1, mask=lastm)
                return 0
            lax.fori_loop(0, nvec, hv, 0)

            def pfx(j, run):
                h = h256[pl.ds(16 * j, 16)]
                cs = plsc.cumsum(h)
                h256[pl.ds(16 * j, 16)] = run + cs - h
                return run + jnp.max(cs)
            lax.fori_loop(0, 16, pfx, jnp.int32(0))

            def sv(v, _):
                k = kin[pl.ds(16 * v, 16)]
                idx = iin[pl.ds(16 * v, 16)]
                d = ((plsc.bitcast(k, jnp.uint32) >> sh)
                     & jnp.uint32(255)).astype(jnp.int32)
                occ, lastm = plsc.scan_count(d)
                occ0 = occ - bias
                base = plsc.load_gather(h256, [d])
                dest = base + occ0
                plsc.store_scatter(kout, [dest], k)
                plsc.store_scatter(iout, [dest], idx)
                plsc.addupdate_scatter(h256, [d], occ0 + 1, mask=lastm)
                return 0
            lax.fori_loop(0, nvec, sv, 0)

        def dv(v, _):
            k = plsc.bitcast(k0[pl.ds(16 * v, 16)] ^ np.int32(-1), jnp.uint32)
            val = _key_to_f32(k)
            scoreb[pl.ds(16 * v, 16)] = 1.0 / (1.0 + jnp.exp(-val))
            return 0
        lax.fori_loop(0, NDEC // 16, dv, 0)

        for cc in range(8):
            def gi(u, _):
                idx = i0[pl.ds(640 * cc + 16 * u, 16)]
                qf = (idx.astype(jnp.float32) * jnp.float32(1.0 / 90.0)
                      + jnp.float32(0.5))
                q = qf.astype(jnp.int32)
                r = idx - q * 90
                q = jnp.minimum(jnp.where(r < 0, q - 1, q), N - 1)
                gidc[pl.ds(16 * u, 16)] = q + N * b
                return 0
            lax.fori_loop(0, 40, gi, 0)
            pltpu.sync_copy(comb_hbm.at[gidc], combrow)

            def dec(u, _):
                rid = iota + 16 * u
                c0 = jnp.zeros((16,), jnp.int32)
                ty = plsc.load_gather(combrow, [rid, c0])
                tx = plsc.load_gather(combrow, [rid, c0 + 1])
                th = plsc.load_gather(combrow, [rid, c0 + 2])
                tw = plsc.load_gather(combrow, [rid, c0 + 3])
                a0 = plsc.load_gather(combrow, [rid, c0 + 4])
                a1 = plsc.load_gather(combrow, [rid, c0 + 5])
                a2 = plsc.load_gather(combrow, [rid, c0 + 6])
                a3 = plsc.load_gather(combrow, [rid, c0 + 7])
                yca = (a0 + a2) * 0.5
                xca = (a1 + a3) * 0.5
                ha = a2 - a0
                wa = a3 - a1
                ww = jnp.exp(tw) * wa
                hh = jnp.exp(th) * ha
                yc = ty * ha + yca
                xc = tx * wa + xca
                y1 = yc - hh * 0.5
                x1 = xc - ww * 0.5
                y2 = yc + hh * 0.5
                x2 = xc + ww * 0.5
                o = pl.ds(640 * cc + 16 * u, 16)
                y1b[o] = y1
                x1b[o] = x1
                y2b[o] = y2
                x2b[o] = x2
                return 0
            lax.fori_loop(0, 40, dec, 0)

        def initacc(j, _):
            far = jnp.full((16,), 3e8, jnp.float32)
            ay1[pl.ds(16 * j, 16)] = far
            ax1[pl.ds(16 * j, 16)] = far
            ay2[pl.ds(16 * j, 16)] = far
            ax2[pl.ds(16 * j, 16)] = far
            aar[pl.ds(16 * j, 16)] = jnp.zeros((16,), jnp.float32)
            return 0
        lax.fori_loop(0, 7, initacc, 0)

        def initout(j, _):
            outb[pl.ds(16 * j, 16)] = jnp.zeros((16,), jnp.float32)
            return 0
        lax.fori_loop(0, 38, initout, 0)

        scale = jnp.max(plsc.load_gather(
            sclv, [jnp.full((16,), b, jnp.int32)]))
        bound = jnp.minimum(jnp.int32(K_TOP), total)

        def nms_cond(carry):
            i, n = carry
            return jnp.logical_and(n < MAX_DET, i < bound)

        def nms_body(carry):
            i, n = carry
            cy1 = y1b[pl.ds(i, 16)][0]
            cx1 = x1b[pl.ds(i, 16)][0]
            cy2 = y2b[pl.ds(i, 16)][0]
            cx2 = x2b[pl.ds(i, 16)][0]
            car = (cy2 - cy1) * (cx2 - cx1)

            def iou_a(j, mx):
                o = pl.ds(16 * j, 16)
                yA = jnp.maximum(jnp.full((16,), cy1), ay1[o])
                xA = jnp.maximum(jnp.full((16,), cx1), ax1[o])
                yB = jnp.minimum(jnp.full((16,), cy2), ay2[o])
                xB = jnp.minimum(jnp.full((16,), cx2), ax2[o])
                inter = jnp.maximum(yB - yA, 0.0) * jnp.maximum(xB - xA, 0.0)
                iou = inter / (car + aar[o] - inter + 1e-8)
                return jnp.maximum(mx, jnp.max(iou))

            mx = lax.fori_loop(0, (n + 15) // 16, iou_a, jnp.float32(0.0))
            keep = mx <= IOU_THR

            @pl.when(keep)
            def _():
                _store1(ay1, n, cy1, iota)
                _store1(ax1, n, cx1, iota)
                _store1(ay2, n, cy2, iota)
                _store1(ax2, n, cx2, iota)
                _store1(aar, n, car, iota)
                idx_s = i0[pl.ds(i, 16)][0]
                qf = (idx_s.astype(jnp.float32) * jnp.float32(1.0 / 90.0)
                      + jnp.float32(0.5))
                q = qf.astype(jnp.int32)
                rr = idx_s - q * 90
                q = jnp.where(rr < 0, q - 1, q)
                cls = idx_s - 90 * q
                _store1(outb, n * 6 + 0, cy1 * scale, iota)
                _store1(outb, n * 6 + 1, cx1 * scale, iota)
                _store1(outb, n * 6 + 2, cy2 * scale, iota)
                _store1(outb, n * 6 + 3, cx2 * scale, iota)
                _store1(outb, n * 6 + 4, scoreb[pl.ds(i, 16)][0], iota)
                _store1(outb, n * 6 + 5, (cls + 1).astype(jnp.float32), iota)

            return i + 1, n + keep.astype(jnp.int32)

        lax.while_loop(nms_cond, nms_body, (jnp.int32(0), jnp.int32(0)))
        pltpu.sync_copy(outb.at[pl.ds(0, 600)], out_hbm.at[b])


@jax.jit
def kernel(cls_out, box_out, anchors, image_scales):
    mesh = plsc.VectorSubcoreMesh(core_axis_name="c", subcore_axis_name="s")
    f = pl.kernel(
        _sc_body,
        out_type=jax.ShapeDtypeStruct((B, 600), jnp.float32),
        mesh=mesh,
        compiler_params=pltpu.CompilerParams(needs_layout_passes=False,
                                             use_tc_tiling_on_sc=False),
        scratch_types=[
            pltpu.VMEM((CH,), jnp.float32),
            pltpu.VMEM((CH,), jnp.float32),
            pltpu.SemaphoreType.DMA,
            pltpu.SemaphoreType.DMA,
            pltpu.VMEM((8192,), jnp.int32),
            pltpu.VMEM((16, 256), jnp.int32),
            pltpu.VMEM((256,), jnp.int32),
            pltpu.VMEM((32,), jnp.int32),
            pltpu.VMEM((512,), jnp.int32),
            pltpu.VMEM((16,), jnp.int32),
            pltpu.VMEM((CAP_T,), jnp.int32),
            pltpu.VMEM((CAP_T,), jnp.int32),
            pltpu.VMEM((16,), jnp.int32),
            pltpu.VMEM((16, 16), jnp.int32),
            pltpu.VMEM((CAP_T,), jnp.int32),
            pltpu.VMEM((CAP_T,), jnp.int32),
            pltpu.VMEM((CAP,), jnp.int32),
            pltpu.VMEM((CAP,), jnp.int32),
            pltpu.VMEM((CAP,), jnp.int32),
            pltpu.VMEM((CAP,), jnp.int32),
            pltpu.VMEM((256,), jnp.int32),
            pltpu.VMEM((640,), jnp.int32),
            pltpu.VMEM((NDEC,), jnp.float32),
            pltpu.VMEM((NDEC,), jnp.float32),
            pltpu.VMEM((NDEC,), jnp.float32),
            pltpu.VMEM((NDEC,), jnp.float32),
            pltpu.VMEM((NDEC,), jnp.float32),
            pltpu.VMEM((640, 8), jnp.float32),
            pltpu.VMEM((16,), jnp.float32),
            pltpu.VMEM((112,), jnp.float32),
            pltpu.VMEM((112,), jnp.float32),
            pltpu.VMEM((112,), jnp.float32),
            pltpu.VMEM((112,), jnp.float32),
            pltpu.VMEM((112,), jnp.float32),
            pltpu.VMEM((608,), jnp.float32),
            pltpu.VMEM_SHARED((16, 8192), jnp.int32),
            pltpu.VMEM_SHARED((8192,), jnp.int32),
            pltpu.VMEM_SHARED((512,), jnp.int32),
            pltpu.VMEM_SHARED((4, 16, CAP_T), jnp.int32),
            pltpu.VMEM_SHARED((4, 16, CAP_T), jnp.int32),
            pltpu.VMEM_SHARED((4, 16, 16), jnp.int32),
        ],
    )
    cls_flat = cls_out.reshape(-1)
    comb = jnp.concatenate(
        [box_out, jnp.broadcast_to(anchors[None], (B, N, 4))], axis=2
    ).reshape(B * N, 8)
    scl16 = jnp.pad(image_scales, (0, 16 - B))
    out = f(cls_flat, comb, scl16)
    return out.reshape(B, MAX_DET, 6)
